# baseline (speedup 1.0000x reference)
import jax
import jax.numpy as jnp
from jax.experimental import pallas as pl

_NEG_INF = float("-inf")


def _top2(gt, eidx):
    e = gt.shape[0]
    m1 = jnp.max(gt, axis=0, keepdims=True)
    i1 = jnp.min(jnp.where(gt == m1, eidx, e), axis=0, keepdims=True)
    g2 = jnp.where(eidx == i1, _NEG_INF, gt)
    m2 = jnp.max(g2, axis=0, keepdims=True)
    i2 = jnp.min(jnp.where(g2 == m2, eidx, e), axis=0, keepdims=True)
    e2 = jnp.exp(m2 - m1)
    d = 1.0 / (1.0 + e2)
    return (jnp.concatenate([i1, i2], axis=0),
            jnp.concatenate([d, e2 * d], axis=0))


def _gate_body(xa_ref, xb_ref, wt_ref, b_ref,
               idxa_ref, scorea_ref, idxb_ref, scoreb_ref):
    wt = wt_ref[...]
    bias = b_ref[...]
    ga = jnp.dot(xa_ref[...], wt, preferred_element_type=jnp.float32)
    gb = jnp.dot(xb_ref[...], wt, preferred_element_type=jnp.float32)
    gta = ga.T + bias
    gtb = gb.T + bias
    eidx = jax.lax.broadcasted_iota(jnp.int32, gta.shape, 0)
    idxa_ref[...], scorea_ref[...] = _top2(gta, eidx)
    idxb_ref[...], scoreb_ref[...] = _top2(gtb, eidx)


def kernel(inp, W, b):
    m, dm = inp.shape
    e = W.shape[0]
    bm = 4096
    h = m // 2
    grid = (h // bm,)
    wt = W.T
    b2 = b.reshape(e, 1)
    xa = inp[:h]
    xb = inp[h:]
    out = pl.pallas_call(
        _gate_body,
        grid=grid,
        in_specs=[
            pl.BlockSpec((bm, dm), lambda i: (i, 0)),
            pl.BlockSpec((bm, dm), lambda i: (i, 0)),
            pl.BlockSpec((dm, e), lambda i: (0, 0)),
            pl.BlockSpec((e, 1), lambda i: (0, 0)),
        ],
        out_specs=[
            pl.BlockSpec((2, bm), lambda i: (0, i)),
            pl.BlockSpec((2, bm), lambda i: (0, i)),
            pl.BlockSpec((2, bm), lambda i: (0, i)),
            pl.BlockSpec((2, bm), lambda i: (0, i)),
        ],
        out_shape=[
            jax.ShapeDtypeStruct((2, h), jnp.int32),
            jax.ShapeDtypeStruct((2, h), jnp.float32),
            jax.ShapeDtypeStruct((2, h), jnp.int32),
            jax.ShapeDtypeStruct((2, h), jnp.float32),
        ],
    )(xa, xb, wt, b2)
    idx_t = jnp.concatenate([out[0], out[2]], axis=1)
    score_t = jnp.concatenate([out[1], out[3]], axis=1)
    return idx_t.T, score_t.T


# speedup vs baseline: 2.4956x; 2.4956x over previous
import jax
import jax.numpy as jnp
from jax.experimental import pallas as pl

_NEG_INF = float("-inf")


def _top2(gt, eidx):
    e = gt.shape[0]
    m1 = jnp.max(gt, axis=0, keepdims=True)
    i1 = jnp.min(jnp.where(gt == m1, eidx, e), axis=0, keepdims=True)
    g2 = jnp.where(eidx == i1, _NEG_INF, gt)
    m2 = jnp.max(g2, axis=0, keepdims=True)
    i2 = jnp.min(jnp.where(g2 == m2, eidx, e), axis=0, keepdims=True)
    e2 = jnp.exp(m2 - m1)
    d = 1.0 / (1.0 + e2)
    return (jnp.concatenate([i1, i2], axis=0),
            jnp.concatenate([d, e2 * d], axis=0))


def _gate_body(xa_ref, xb_ref, wt_ref, b_ref,
               idxa_ref, scorea_ref, idxb_ref, scoreb_ref):
    wt = wt_ref[...]
    bias = b_ref[...]
    ga = jnp.dot(xa_ref[...], wt, preferred_element_type=jnp.float32)
    gb = jnp.dot(xb_ref[...], wt, preferred_element_type=jnp.float32)
    gta = ga.T + bias
    gtb = gb.T + bias
    eidx = jax.lax.broadcasted_iota(jnp.int32, gta.shape, 0)
    idxa_ref[...], scorea_ref[...] = _top2(gta, eidx)
    idxb_ref[...], scoreb_ref[...] = _top2(gtb, eidx)


def kernel(inp, W, b):
    m, dm = inp.shape
    e = W.shape[0]
    bm = 4096
    h = m // 2
    grid = (h // bm,)
    wt = W.T
    b2 = b.reshape(e, 1)
    nb = h // bm
    out = pl.pallas_call(
        _gate_body,
        grid=grid,
        in_specs=[
            pl.BlockSpec((bm, dm), lambda i: (i, 0)),
            pl.BlockSpec((bm, dm), lambda i: (i + nb, 0)),
            pl.BlockSpec((dm, e), lambda i: (0, 0)),
            pl.BlockSpec((e, 1), lambda i: (0, 0)),
        ],
        out_specs=[
            pl.BlockSpec((2, bm), lambda i: (0, i)),
            pl.BlockSpec((2, bm), lambda i: (0, i)),
            pl.BlockSpec((2, bm), lambda i: (0, i)),
            pl.BlockSpec((2, bm), lambda i: (0, i)),
        ],
        out_shape=[
            jax.ShapeDtypeStruct((2, h), jnp.int32),
            jax.ShapeDtypeStruct((2, h), jnp.float32),
            jax.ShapeDtypeStruct((2, h), jnp.int32),
            jax.ShapeDtypeStruct((2, h), jnp.float32),
        ],
    )(inp, inp, wt, b2)
    idx_t = jnp.concatenate([out[0], out[2]], axis=1)
    score_t = jnp.concatenate([out[1], out[3]], axis=1)
    return idx_t.T, score_t.T
